# baseline (speedup 1.0000x reference)
import functools

import jax
import jax.numpy as jnp
from jax import lax
from jax.experimental import pallas as pl
from jax.experimental.pallas import tpu as pltpu
from jax.experimental.pallas import tpu_sc as plsc

N = 10000
NP = 10240
E = 640000
D = 128
G = 64
CH = 128
NSC = 2
NT = 16
CPT = 160
IBLK = 16
ECHUNKS = NSC * NT * CPT
EP = ECHUNKS * CH
RPT = NP // NT
BLK = 1280
NPCH = NP // CH
PSPAN = 8

_Z16 = functools.partial(jnp.zeros, (16,), jnp.float32)
_O16 = functools.partial(jnp.ones, (16,), jnp.float32)

_mesh = plsc.VectorSubcoreMesh(core_axis_name="c", subcore_axis_name="s")


def _lrelu(x):
    return jnp.where(x >= 0, x, 0.01 * x)


@functools.partial(
    pl.kernel,
    out_type=jax.ShapeDtypeStruct((NSC * NP,), jnp.float32),
    mesh=_mesh,
    scratch_types=[
        pltpu.VMEM((CPT, CH), jnp.int32),
        pltpu.VMEM((CH,), jnp.float32),
        pltpu.VMEM((RPT,), jnp.float32),
        pltpu.VMEM_SHARED((NP,), jnp.float32),
    ],
)
def _hist(dst_hbm, out_hbm, dstv, onesv, zv, acc):
    c = lax.axis_index("c")
    s = lax.axis_index("s")
    for k in range(CH // 16):
        onesv[pl.ds(k * 16, 16)] = _O16()

    def _z(i, _):
        zv[pl.ds(i * 16, 16)] = _Z16()
        return 0

    lax.fori_loop(0, RPT // 16, _z, 0)
    pltpu.sync_copy(zv, acc.at[pl.ds(s * RPT, RPT)])
    plsc.subcore_barrier()

    base = c * (NT * CPT) + s * CPT
    pltpu.sync_copy(dst_hbm.at[pl.ds(base, CPT)], dstv)

    def _b(j, _):
        pltpu.sync_copy(onesv, acc.at[dstv.at[j]], add=True)
        return 0

    lax.fori_loop(0, CPT, _b, 0)
    plsc.subcore_barrier()
    pltpu.sync_copy(acc.at[pl.ds(s * RPT, RPT)], zv)
    pltpu.sync_copy(zv, out_hbm.at[pl.ds(c * NP + s * RPT, RPT)])


@functools.partial(
    pl.kernel,
    out_type=jax.ShapeDtypeStruct((NSC, NP, D), jnp.float32),
    mesh=_mesh,
    scratch_types=[
        pltpu.VMEM((IBLK, CH), jnp.int32),
        pltpu.VMEM((IBLK, CH), jnp.int32),
        pltpu.VMEM((CH, D), jnp.float32),
        pltpu.VMEM((CH, D), jnp.float32),
        pltpu.SemaphoreType.DMA,
        pltpu.SemaphoreType.DMA,
        pltpu.VMEM_SHARED((NP, D), jnp.float32),
    ],
)
def _edge(h_hbm, src_hbm, dst_hbm, out_hbm, srcv, dstv, buf0, buf1, gsem, ssem, acc):
    c = lax.axis_index("c")
    s = lax.axis_index("s")
    bufs = (buf0, buf1)

    def _z(i, _):
        for k in range(D // 16):
            buf0[i, pl.ds(k * 16, 16)] = _Z16()
        return 0

    lax.fori_loop(0, CH, _z, 0)
    for r in range(RPT // CH):
        pltpu.sync_copy(buf0, acc.at[pl.ds(s * RPT + r * CH, CH)])
    plsc.subcore_barrier()

    base = s * (2 * CPT)

    def _blk(b, _):
        pltpu.sync_copy(src_hbm.at[pl.ds(base + b * IBLK, IBLK)], srcv)
        pltpu.sync_copy(dst_hbm.at[pl.ds(base + b * IBLK, IBLK)], dstv)
        dg = [None] * IBLK
        dsc = [None] * IBLK
        dg[0] = pltpu.async_copy(h_hbm.at[srcv.at[0]], buf0, gsem)
        for j in range(IBLK):
            bj = bufs[j % 2]
            dg[j].wait()
            if j >= 1:
                dsc[j - 1].wait()
            if j + 1 < IBLK:
                dg[j + 1] = pltpu.async_copy(
                    h_hbm.at[srcv.at[j + 1]], bufs[(j + 1) % 2], gsem)
            dsc[j] = pltpu.async_copy(bj, acc.at[dstv.at[j]], ssem, add=True)
        dsc[IBLK - 1].wait()
        return 0

    lax.fori_loop(0, jnp.where(c == 0, 2 * CPT // IBLK, 0), _blk, 0)
    plsc.subcore_barrier()
    pltpu.sync_copy(acc.at[pl.ds(s * RPT, RPT)], out_hbm.at[c, pl.ds(s * RPT, RPT)])


@functools.partial(
    pl.kernel,
    out_type=(
        jax.ShapeDtypeStruct((NSC, 2 * G, D), jnp.float32),
        jax.ShapeDtypeStruct((NSC * 2 * G,), jnp.float32),
    ),
    mesh=_mesh,
    scratch_types=[
        pltpu.VMEM((PSPAN, CH), jnp.int32),
        pltpu.VMEM((CH, D), jnp.float32),
        pltpu.VMEM((CH,), jnp.float32),
        pltpu.VMEM((16,), jnp.float32),
        pltpu.VMEM_SHARED((2 * G, D), jnp.float32),
        pltpu.VMEM_SHARED((2 * G,), jnp.float32),
    ],
)
def _pool(xh_hbm, b_hbm, outs_hbm, outc_hbm, bidx, buf, onesv, zv, accs, accc):
    c = lax.axis_index("c")
    s = lax.axis_index("s")

    def _z(i, _):
        for k in range(D // 16):
            buf[i, pl.ds(k * 16, 16)] = _Z16()
        return 0

    lax.fori_loop(0, CH, _z, 0)
    for k in range(CH // 16):
        onesv[pl.ds(k * 16, 16)] = _O16()
    zv[pl.ds(0, 16)] = _Z16()
    rows = (2 * G) // NT
    pltpu.sync_copy(buf.at[pl.ds(0, rows)], accs.at[pl.ds(s * rows, rows)])

    @pl.when(s < 8)
    def _zc():
        pltpu.sync_copy(zv, accc.at[pl.ds(s * 16, 16)])

    plsc.subcore_barrier()

    span = c * 5 + s
    active = s < 5
    base = jnp.where(active, span * PSPAN, 0)
    pltpu.sync_copy(b_hbm.at[pl.ds(base, PSPAN)], bidx)

    def _b(k, _):
        pltpu.sync_copy(xh_hbm.at[pl.ds((base + k) * CH, CH)], buf)
        pltpu.sync_copy(buf, accs.at[bidx.at[k]], add=True)
        pltpu.sync_copy(onesv, accc.at[bidx.at[k]], add=True)
        return 0

    lax.fori_loop(0, jnp.where(active, PSPAN, 0), _b, 0)
    plsc.subcore_barrier()
    pltpu.sync_copy(accs.at[pl.ds(s * rows, rows)], outs_hbm.at[c, pl.ds(s * rows, rows)])

    @pl.when(s < 8)
    def _wc():
        pltpu.sync_copy(accc.at[pl.ds(s * 16, 16)], zv)
        pltpu.sync_copy(zv, outc_hbm.at[pl.ds(c * 2 * G + s * 16, 16)])


def _prep_body(degp_ref, x_ref, w_ref, db_ref, h_ref):
    deg = degp_ref[:, 0:1] + degp_ref[:, 1:2] + 1.0
    db = jnp.broadcast_to(lax.rsqrt(deg), (BLK, D))
    db_ref[...] = db
    h_ref[...] = jnp.dot(x_ref[...] * db, w_ref[...],
                         preferred_element_type=jnp.float32)


_prep = pl.pallas_call(
    _prep_body,
    grid=(NP // BLK,),
    in_specs=[
        pl.BlockSpec((BLK, 2), lambda i: (i, 0)),
        pl.BlockSpec((BLK, D), lambda i: (i, 0)),
        pl.BlockSpec((D, D), lambda i: (0, 0)),
    ],
    out_specs=[pl.BlockSpec((BLK, D), lambda i: (i, 0))] * 2,
    out_shape=[jax.ShapeDtypeStruct((NP, D), jnp.float32)] * 2,
)


def _mid_body(p0_ref, p1_ref, h_ref, db_ref, b_ref, w_ref, out_ref):
    db = db_ref[...]
    sv = db * (p0_ref[0] + p1_ref[0] + h_ref[...]) + b_ref[...]
    out_ref[...] = jnp.dot(db * _lrelu(sv), w_ref[...],
                           preferred_element_type=jnp.float32)


_mid = pl.pallas_call(
    _mid_body,
    grid=(NP // BLK,),
    in_specs=[
        pl.BlockSpec((1, BLK, D), lambda i: (0, i, 0)),
        pl.BlockSpec((1, BLK, D), lambda i: (1, i, 0)),
        pl.BlockSpec((BLK, D), lambda i: (i, 0)),
        pl.BlockSpec((BLK, D), lambda i: (i, 0)),
        pl.BlockSpec((1, D), lambda i: (0, 0)),
        pl.BlockSpec((D, D), lambda i: (0, 0)),
    ],
    out_specs=pl.BlockSpec((BLK, D), lambda i: (i, 0)),
    out_shape=jax.ShapeDtypeStruct((NP, D), jnp.float32),
)


def _last_body(p0_ref, p1_ref, h_ref, db_ref, b_ref, wl_ref, bl_ref, out_ref):
    sv = db_ref[...] * (p0_ref[0] + p1_ref[0] + h_ref[...]) + b_ref[...]
    out_ref[...] = _lrelu(jnp.dot(sv, wl_ref[...],
                                  preferred_element_type=jnp.float32) + bl_ref[...])


_last = pl.pallas_call(
    _last_body,
    grid=(NP // BLK,),
    in_specs=[
        pl.BlockSpec((1, BLK, D), lambda i: (0, i, 0)),
        pl.BlockSpec((1, BLK, D), lambda i: (1, i, 0)),
        pl.BlockSpec((BLK, D), lambda i: (i, 0)),
        pl.BlockSpec((BLK, D), lambda i: (i, 0)),
        pl.BlockSpec((1, D), lambda i: (0, 0)),
        pl.BlockSpec((D, D), lambda i: (0, 0)),
        pl.BlockSpec((1, D), lambda i: (0, 0)),
    ],
    out_specs=pl.BlockSpec((BLK, D), lambda i: (i, 0)),
    out_shape=jax.ShapeDtypeStruct((NP, D), jnp.float32),
)


def _final_body(s0_ref, s1_ref, ct_ref, wf_ref, bf_ref, out_ref):
    ssum = s0_ref[0] + s1_ref[0]
    csum = ct_ref[:, 0:1] + ct_ref[:, 1:2]
    hm = ssum / jnp.maximum(csum, 1.0)
    r = jnp.sum(hm * wf_ref[...], axis=1, keepdims=True) + bf_ref[...]
    out_ref[...] = r[:G, :]


_final = pl.pallas_call(
    _final_body,
    grid=(1,),
    in_specs=[
        pl.BlockSpec((1, 2 * G, D), lambda i: (0, 0, 0)),
        pl.BlockSpec((1, 2 * G, D), lambda i: (1, 0, 0)),
        pl.BlockSpec((2 * G, 2), lambda i: (0, 0)),
        pl.BlockSpec((1, D), lambda i: (0, 0)),
        pl.BlockSpec((1, 1), lambda i: (0, 0)),
    ],
    out_specs=pl.BlockSpec((G, 1), lambda i: (0, 0)),
    out_shape=jax.ShapeDtypeStruct((G, 1), jnp.float32),
)


def kernel(fts, adj, batch, W1, b1, W2, b2, W3, b3, Wl, bl, Wf, bf):
    src, dst = adj[0], adj[1]
    pad_e = EP - E
    srcp = jnp.concatenate(
        [src, jnp.zeros((pad_e,), jnp.int32)]).reshape(ECHUNKS, CH)
    dstp = jnp.concatenate(
        [dst, jnp.full((pad_e,), N, jnp.int32)]).reshape(ECHUNKS, CH)
    xp = jnp.pad(fts, ((0, NP - N), (0, 0)))
    batchp = jnp.concatenate(
        [batch, jnp.full((NP - N,), G, jnp.int32)]).reshape(NPCH, CH)

    degp = _hist(dstp).reshape(NSC, NP)
    db, h = _prep(degp.T, xp, W1)

    p = _edge(h, srcp, dstp)
    h = _mid(p, p, h, db, b1.reshape(1, D), W2)
    p = _edge(h, srcp, dstp)
    h = _mid(p, p, h, db, b2.reshape(1, D), W3)
    p = _edge(h, srcp, dstp)
    xh = _last(p, p, h, db, b3.reshape(1, D), Wl, bl.reshape(1, D))

    ps, pc = _pool(xh, batchp)
    return _final(ps, ps, pc.reshape(NSC, 2 * G).T,
                  Wf.reshape(1, D), bf.reshape(1, 1))


# speedup vs baseline: 1.3496x; 1.3496x over previous
import functools

import jax
import jax.numpy as jnp
from jax import lax
from jax.experimental import pallas as pl
from jax.experimental.pallas import tpu as pltpu
from jax.experimental.pallas import tpu_sc as plsc

N = 10000
NP = 10240
E = 640000
D = 128
G = 64
CH = 128
NSC = 2
NT = 16
CPT = 160
CPT0 = 224
CPT1 = 96
IBLK = 16
ECHUNKS = NSC * NT * CPT
EP = ECHUNKS * CH
RPT = NP // NT
BLK = 1280
NPCH = NP // CH
PSPAN = 8

_Z16 = functools.partial(jnp.zeros, (16,), jnp.float32)
_O16 = functools.partial(jnp.ones, (16,), jnp.float32)

_mesh = plsc.VectorSubcoreMesh(core_axis_name="c", subcore_axis_name="s")


def _lrelu(x):
    return jnp.where(x >= 0, x, 0.01 * x)


@functools.partial(
    pl.kernel,
    out_type=jax.ShapeDtypeStruct((NSC * NP,), jnp.float32),
    mesh=_mesh,
    scratch_types=[
        pltpu.VMEM((CPT, CH), jnp.int32),
        pltpu.VMEM((CH,), jnp.float32),
        pltpu.VMEM((RPT,), jnp.float32),
        pltpu.VMEM_SHARED((NP,), jnp.float32),
    ],
)
def _hist(dst_hbm, out_hbm, dstv, onesv, zv, acc):
    c = lax.axis_index("c")
    s = lax.axis_index("s")
    for k in range(CH // 16):
        onesv[pl.ds(k * 16, 16)] = _O16()

    def _z(i, _):
        zv[pl.ds(i * 16, 16)] = _Z16()
        return 0

    lax.fori_loop(0, RPT // 16, _z, 0)
    pltpu.sync_copy(zv, acc.at[pl.ds(s * RPT, RPT)])
    plsc.subcore_barrier()

    base = c * (NT * CPT) + s * CPT
    pltpu.sync_copy(dst_hbm.at[pl.ds(base, CPT)], dstv)

    def _b(j, _):
        pltpu.sync_copy(onesv, acc.at[dstv.at[j]], add=True)
        return 0

    lax.fori_loop(0, CPT, _b, 0)
    plsc.subcore_barrier()
    pltpu.sync_copy(acc.at[pl.ds(s * RPT, RPT)], zv)
    pltpu.sync_copy(zv, out_hbm.at[pl.ds(c * NP + s * RPT, RPT)])


@functools.partial(
    pl.kernel,
    out_type=jax.ShapeDtypeStruct((NSC, NP, D), jnp.float32),
    mesh=_mesh,
    scratch_types=[
        pltpu.VMEM((IBLK, CH), jnp.int32),
        pltpu.VMEM((IBLK, CH), jnp.int32),
        pltpu.VMEM((CH, D), jnp.float32),
        pltpu.VMEM((CH, D), jnp.float32),
        pltpu.SemaphoreType.DMA,
        pltpu.SemaphoreType.DMA,
        pltpu.VMEM_SHARED((NP, D), jnp.float32),
    ],
)
def _edge(h_hbm, src_hbm, dst_hbm, out_hbm, srcv, dstv, buf0, buf1, gsem, ssem, acc):
    c = lax.axis_index("c")
    s = lax.axis_index("s")
    bufs = (buf0, buf1)

    def _z(i, _):
        for k in range(D // 16):
            buf0[i, pl.ds(k * 16, 16)] = _Z16()
        return 0

    lax.fori_loop(0, CH, _z, 0)
    for r in range(RPT // CH):
        pltpu.sync_copy(buf0, acc.at[pl.ds(s * RPT + r * CH, CH)])
    plsc.subcore_barrier()

    base = jnp.where(c == 0, s * CPT0, NT * CPT0 + s * CPT1)

    def _blk(b, _):
        pltpu.sync_copy(src_hbm.at[pl.ds(base + b * IBLK, IBLK)], srcv)
        pltpu.sync_copy(dst_hbm.at[pl.ds(base + b * IBLK, IBLK)], dstv)
        dg = [None] * IBLK
        dsc = [None] * IBLK
        dg[0] = pltpu.async_copy(h_hbm.at[srcv.at[0]], buf0, gsem)
        for j in range(IBLK):
            bj = bufs[j % 2]
            dg[j].wait()
            if j >= 1:
                dsc[j - 1].wait()
            if j + 1 < IBLK:
                dg[j + 1] = pltpu.async_copy(
                    h_hbm.at[srcv.at[j + 1]], bufs[(j + 1) % 2], gsem)
            dsc[j] = pltpu.async_copy(bj, acc.at[dstv.at[j]], ssem, add=True)
        dsc[IBLK - 1].wait()
        return 0

    lax.fori_loop(0, jnp.where(c == 0, CPT0 // IBLK, CPT1 // IBLK), _blk, 0)
    plsc.subcore_barrier()
    pltpu.sync_copy(acc.at[pl.ds(s * RPT, RPT)], out_hbm.at[c, pl.ds(s * RPT, RPT)])


@functools.partial(
    pl.kernel,
    out_type=(
        jax.ShapeDtypeStruct((NSC, 2 * G, D), jnp.float32),
        jax.ShapeDtypeStruct((NSC * 2 * G,), jnp.float32),
    ),
    mesh=_mesh,
    scratch_types=[
        pltpu.VMEM((PSPAN, CH), jnp.int32),
        pltpu.VMEM((CH, D), jnp.float32),
        pltpu.VMEM((CH,), jnp.float32),
        pltpu.VMEM((16,), jnp.float32),
        pltpu.VMEM_SHARED((2 * G, D), jnp.float32),
        pltpu.VMEM_SHARED((2 * G,), jnp.float32),
    ],
)
def _pool(xh_hbm, b_hbm, outs_hbm, outc_hbm, bidx, buf, onesv, zv, accs, accc):
    c = lax.axis_index("c")
    s = lax.axis_index("s")

    def _z(i, _):
        for k in range(D // 16):
            buf[i, pl.ds(k * 16, 16)] = _Z16()
        return 0

    lax.fori_loop(0, CH, _z, 0)
    for k in range(CH // 16):
        onesv[pl.ds(k * 16, 16)] = _O16()
    zv[pl.ds(0, 16)] = _Z16()
    rows = (2 * G) // NT
    pltpu.sync_copy(buf.at[pl.ds(0, rows)], accs.at[pl.ds(s * rows, rows)])

    @pl.when(s < 8)
    def _zc():
        pltpu.sync_copy(zv, accc.at[pl.ds(s * 16, 16)])

    plsc.subcore_barrier()

    span = c * 5 + s
    active = s < 5
    base = jnp.where(active, span * PSPAN, 0)
    pltpu.sync_copy(b_hbm.at[pl.ds(base, PSPAN)], bidx)

    def _b(k, _):
        pltpu.sync_copy(xh_hbm.at[pl.ds((base + k) * CH, CH)], buf)
        pltpu.sync_copy(buf, accs.at[bidx.at[k]], add=True)
        pltpu.sync_copy(onesv, accc.at[bidx.at[k]], add=True)
        return 0

    lax.fori_loop(0, jnp.where(active, PSPAN, 0), _b, 0)
    plsc.subcore_barrier()
    pltpu.sync_copy(accs.at[pl.ds(s * rows, rows)], outs_hbm.at[c, pl.ds(s * rows, rows)])

    @pl.when(s < 8)
    def _wc():
        pltpu.sync_copy(accc.at[pl.ds(s * 16, 16)], zv)
        pltpu.sync_copy(zv, outc_hbm.at[pl.ds(c * 2 * G + s * 16, 16)])


def _prep_body(degp_ref, x_ref, w_ref, db_ref, h_ref):
    deg = degp_ref[:, 0:1] + degp_ref[:, 1:2] + 1.0
    db = jnp.broadcast_to(lax.rsqrt(deg), (BLK, D))
    db_ref[...] = db
    h_ref[...] = jnp.dot(x_ref[...] * db, w_ref[...],
                         preferred_element_type=jnp.float32)


_prep = pl.pallas_call(
    _prep_body,
    grid=(NP // BLK,),
    in_specs=[
        pl.BlockSpec((BLK, 2), lambda i: (i, 0)),
        pl.BlockSpec((BLK, D), lambda i: (i, 0)),
        pl.BlockSpec((D, D), lambda i: (0, 0)),
    ],
    out_specs=[pl.BlockSpec((BLK, D), lambda i: (i, 0))] * 2,
    out_shape=[jax.ShapeDtypeStruct((NP, D), jnp.float32)] * 2,
)


def _mid_body(p0_ref, p1_ref, h_ref, db_ref, b_ref, w_ref, out_ref):
    db = db_ref[...]
    sv = db * (p0_ref[0] + p1_ref[0] + h_ref[...]) + b_ref[...]
    out_ref[...] = jnp.dot(db * _lrelu(sv), w_ref[...],
                           preferred_element_type=jnp.float32)


_mid = pl.pallas_call(
    _mid_body,
    grid=(NP // BLK,),
    in_specs=[
        pl.BlockSpec((1, BLK, D), lambda i: (0, i, 0)),
        pl.BlockSpec((1, BLK, D), lambda i: (1, i, 0)),
        pl.BlockSpec((BLK, D), lambda i: (i, 0)),
        pl.BlockSpec((BLK, D), lambda i: (i, 0)),
        pl.BlockSpec((1, D), lambda i: (0, 0)),
        pl.BlockSpec((D, D), lambda i: (0, 0)),
    ],
    out_specs=pl.BlockSpec((BLK, D), lambda i: (i, 0)),
    out_shape=jax.ShapeDtypeStruct((NP, D), jnp.float32),
)


def _last_body(p0_ref, p1_ref, h_ref, db_ref, b_ref, wl_ref, bl_ref, out_ref):
    sv = db_ref[...] * (p0_ref[0] + p1_ref[0] + h_ref[...]) + b_ref[...]
    out_ref[...] = _lrelu(jnp.dot(sv, wl_ref[...],
                                  preferred_element_type=jnp.float32) + bl_ref[...])


_last = pl.pallas_call(
    _last_body,
    grid=(NP // BLK,),
    in_specs=[
        pl.BlockSpec((1, BLK, D), lambda i: (0, i, 0)),
        pl.BlockSpec((1, BLK, D), lambda i: (1, i, 0)),
        pl.BlockSpec((BLK, D), lambda i: (i, 0)),
        pl.BlockSpec((BLK, D), lambda i: (i, 0)),
        pl.BlockSpec((1, D), lambda i: (0, 0)),
        pl.BlockSpec((D, D), lambda i: (0, 0)),
        pl.BlockSpec((1, D), lambda i: (0, 0)),
    ],
    out_specs=pl.BlockSpec((BLK, D), lambda i: (i, 0)),
    out_shape=jax.ShapeDtypeStruct((NP, D), jnp.float32),
)


def _final_body(s0_ref, s1_ref, ct_ref, wf_ref, bf_ref, out_ref):
    ssum = s0_ref[0] + s1_ref[0]
    csum = ct_ref[:, 0:1] + ct_ref[:, 1:2]
    hm = ssum / jnp.maximum(csum, 1.0)
    r = jnp.sum(hm * wf_ref[...], axis=1, keepdims=True) + bf_ref[...]
    out_ref[...] = r[:G, :]


_final = pl.pallas_call(
    _final_body,
    grid=(1,),
    in_specs=[
        pl.BlockSpec((1, 2 * G, D), lambda i: (0, 0, 0)),
        pl.BlockSpec((1, 2 * G, D), lambda i: (1, 0, 0)),
        pl.BlockSpec((2 * G, 2), lambda i: (0, 0)),
        pl.BlockSpec((1, D), lambda i: (0, 0)),
        pl.BlockSpec((1, 1), lambda i: (0, 0)),
    ],
    out_specs=pl.BlockSpec((G, 1), lambda i: (0, 0)),
    out_shape=jax.ShapeDtypeStruct((G, 1), jnp.float32),
)


def kernel(fts, adj, batch, W1, b1, W2, b2, W3, b3, Wl, bl, Wf, bf):
    src, dst = adj[0], adj[1]
    pad_e = EP - E
    srcp = jnp.concatenate(
        [src, jnp.zeros((pad_e,), jnp.int32)]).reshape(ECHUNKS, CH)
    dstp = jnp.concatenate(
        [dst, jnp.full((pad_e,), N, jnp.int32)]).reshape(ECHUNKS, CH)
    xp = jnp.pad(fts, ((0, NP - N), (0, 0)))
    batchp = jnp.concatenate(
        [batch, jnp.full((NP - N,), G, jnp.int32)]).reshape(NPCH, CH)

    degp = _hist(dstp).reshape(NSC, NP)
    db, h = _prep(degp.T, xp, W1)

    p = _edge(h, srcp, dstp)
    h = _mid(p, p, h, db, b1.reshape(1, D), W2)
    p = _edge(h, srcp, dstp)
    h = _mid(p, p, h, db, b2.reshape(1, D), W3)
    p = _edge(h, srcp, dstp)
    xh = _last(p, p, h, db, b3.reshape(1, D), Wl, bl.reshape(1, D))

    ps, pc = _pool(xh, batchp)
    return _final(ps, ps, pc.reshape(NSC, 2 * G).T,
                  Wf.reshape(1, D), bf.reshape(1, 1))


# speedup vs baseline: 1.5688x; 1.1624x over previous
import functools

import jax
import jax.numpy as jnp
from jax import lax
from jax.experimental import pallas as pl
from jax.experimental.pallas import tpu as pltpu
from jax.experimental.pallas import tpu_sc as plsc

N = 10000
NP = 10240
E = 640000
D = 128
G = 64
CH = 128
NSC = 2
NT = 16
CPT = 160
CPT0 = 288
CPT1 = 32
IBLK = 16
ECHUNKS = NSC * NT * CPT
EP = ECHUNKS * CH
RPT = NP // NT
BLK = 1280
NPCH = NP // CH
PSPAN = 8

_Z16 = functools.partial(jnp.zeros, (16,), jnp.float32)
_O16 = functools.partial(jnp.ones, (16,), jnp.float32)

_mesh = plsc.VectorSubcoreMesh(core_axis_name="c", subcore_axis_name="s")


def _lrelu(x):
    return jnp.where(x >= 0, x, 0.01 * x)


@functools.partial(
    pl.kernel,
    out_type=jax.ShapeDtypeStruct((NSC * NP,), jnp.float32),
    mesh=_mesh,
    scratch_types=[
        pltpu.VMEM((CPT, CH), jnp.int32),
        pltpu.VMEM((CH,), jnp.float32),
        pltpu.VMEM((RPT,), jnp.float32),
        pltpu.VMEM_SHARED((NP,), jnp.float32),
    ],
)
def _hist(dst_hbm, out_hbm, dstv, onesv, zv, acc):
    c = lax.axis_index("c")
    s = lax.axis_index("s")
    for k in range(CH // 16):
        onesv[pl.ds(k * 16, 16)] = _O16()

    def _z(i, _):
        zv[pl.ds(i * 16, 16)] = _Z16()
        return 0

    lax.fori_loop(0, RPT // 16, _z, 0)
    pltpu.sync_copy(zv, acc.at[pl.ds(s * RPT, RPT)])
    plsc.subcore_barrier()

    base = c * (NT * CPT) + s * CPT
    pltpu.sync_copy(dst_hbm.at[pl.ds(base, CPT)], dstv)

    def _b(j, _):
        pltpu.sync_copy(onesv, acc.at[dstv.at[j]], add=True)
        return 0

    lax.fori_loop(0, CPT, _b, 0)
    plsc.subcore_barrier()
    pltpu.sync_copy(acc.at[pl.ds(s * RPT, RPT)], zv)
    pltpu.sync_copy(zv, out_hbm.at[pl.ds(c * NP + s * RPT, RPT)])


@functools.partial(
    pl.kernel,
    out_type=jax.ShapeDtypeStruct((NSC, NP, D), jnp.float32),
    mesh=_mesh,
    scratch_types=[
        pltpu.VMEM((IBLK, CH), jnp.int32),
        pltpu.VMEM((IBLK, CH), jnp.int32),
        pltpu.VMEM((CH, D), jnp.float32),
        pltpu.VMEM((CH, D), jnp.float32),
        pltpu.SemaphoreType.DMA,
        pltpu.SemaphoreType.DMA,
        pltpu.VMEM_SHARED((NP, D), jnp.float32),
    ],
)
def _edge(h_hbm, src_hbm, dst_hbm, out_hbm, srcv, dstv, buf0, buf1, gsem, ssem, acc):
    c = lax.axis_index("c")
    s = lax.axis_index("s")
    bufs = (buf0, buf1)

    def _z(i, _):
        for k in range(D // 16):
            buf0[i, pl.ds(k * 16, 16)] = _Z16()
        return 0

    lax.fori_loop(0, CH, _z, 0)
    for r in range(RPT // CH):
        pltpu.sync_copy(buf0, acc.at[pl.ds(s * RPT + r * CH, CH)])
    plsc.subcore_barrier()

    base = jnp.where(c == 0, s * CPT0, NT * CPT0 + s * CPT1)

    def _blk(b, _):
        pltpu.sync_copy(src_hbm.at[pl.ds(base + b * IBLK, IBLK)], srcv)
        pltpu.sync_copy(dst_hbm.at[pl.ds(base + b * IBLK, IBLK)], dstv)
        dg = [None] * IBLK
        dsc = [None] * IBLK
        dg[0] = pltpu.async_copy(h_hbm.at[srcv.at[0]], buf0, gsem)
        for j in range(IBLK):
            bj = bufs[j % 2]
            dg[j].wait()
            if j >= 1:
                dsc[j - 1].wait()
            if j + 1 < IBLK:
                dg[j + 1] = pltpu.async_copy(
                    h_hbm.at[srcv.at[j + 1]], bufs[(j + 1) % 2], gsem)
            dsc[j] = pltpu.async_copy(bj, acc.at[dstv.at[j]], ssem, add=True)
        dsc[IBLK - 1].wait()
        return 0

    lax.fori_loop(0, jnp.where(c == 0, CPT0 // IBLK, CPT1 // IBLK), _blk, 0)
    plsc.subcore_barrier()
    pltpu.sync_copy(acc.at[pl.ds(s * RPT, RPT)], out_hbm.at[c, pl.ds(s * RPT, RPT)])


@functools.partial(
    pl.kernel,
    out_type=(
        jax.ShapeDtypeStruct((NSC, 2 * G, D), jnp.float32),
        jax.ShapeDtypeStruct((NSC * 2 * G,), jnp.float32),
    ),
    mesh=_mesh,
    scratch_types=[
        pltpu.VMEM((PSPAN, CH), jnp.int32),
        pltpu.VMEM((CH, D), jnp.float32),
        pltpu.VMEM((CH,), jnp.float32),
        pltpu.VMEM((16,), jnp.float32),
        pltpu.VMEM_SHARED((2 * G, D), jnp.float32),
        pltpu.VMEM_SHARED((2 * G,), jnp.float32),
    ],
)
def _pool(xh_hbm, b_hbm, outs_hbm, outc_hbm, bidx, buf, onesv, zv, accs, accc):
    c = lax.axis_index("c")
    s = lax.axis_index("s")

    def _z(i, _):
        for k in range(D // 16):
            buf[i, pl.ds(k * 16, 16)] = _Z16()
        return 0

    lax.fori_loop(0, CH, _z, 0)
    for k in range(CH // 16):
        onesv[pl.ds(k * 16, 16)] = _O16()
    zv[pl.ds(0, 16)] = _Z16()
    rows = (2 * G) // NT
    pltpu.sync_copy(buf.at[pl.ds(0, rows)], accs.at[pl.ds(s * rows, rows)])

    @pl.when(s < 8)
    def _zc():
        pltpu.sync_copy(zv, accc.at[pl.ds(s * 16, 16)])

    plsc.subcore_barrier()

    span = c * 5 + s
    active = s < 5
    base = jnp.where(active, span * PSPAN, 0)
    pltpu.sync_copy(b_hbm.at[pl.ds(base, PSPAN)], bidx)

    def _b(k, _):
        pltpu.sync_copy(xh_hbm.at[pl.ds((base + k) * CH, CH)], buf)
        pltpu.sync_copy(buf, accs.at[bidx.at[k]], add=True)
        pltpu.sync_copy(onesv, accc.at[bidx.at[k]], add=True)
        return 0

    lax.fori_loop(0, jnp.where(active, PSPAN, 0), _b, 0)
    plsc.subcore_barrier()
    pltpu.sync_copy(accs.at[pl.ds(s * rows, rows)], outs_hbm.at[c, pl.ds(s * rows, rows)])

    @pl.when(s < 8)
    def _wc():
        pltpu.sync_copy(accc.at[pl.ds(s * 16, 16)], zv)
        pltpu.sync_copy(zv, outc_hbm.at[pl.ds(c * 2 * G + s * 16, 16)])


def _prep_body(degp_ref, x_ref, w_ref, db_ref, h_ref):
    deg = degp_ref[:, 0:1] + degp_ref[:, 1:2] + 1.0
    db = jnp.broadcast_to(lax.rsqrt(deg), (BLK, D))
    db_ref[...] = db
    h_ref[...] = jnp.dot(x_ref[...] * db, w_ref[...],
                         preferred_element_type=jnp.float32)


_prep = pl.pallas_call(
    _prep_body,
    grid=(NP // BLK,),
    in_specs=[
        pl.BlockSpec((BLK, 2), lambda i: (i, 0)),
        pl.BlockSpec((BLK, D), lambda i: (i, 0)),
        pl.BlockSpec((D, D), lambda i: (0, 0)),
    ],
    out_specs=[pl.BlockSpec((BLK, D), lambda i: (i, 0))] * 2,
    out_shape=[jax.ShapeDtypeStruct((NP, D), jnp.float32)] * 2,
)


def _mid_body(p0_ref, p1_ref, h_ref, db_ref, b_ref, w_ref, out_ref):
    db = db_ref[...]
    sv = db * (p0_ref[0] + p1_ref[0] + h_ref[...]) + b_ref[...]
    out_ref[...] = jnp.dot(db * _lrelu(sv), w_ref[...],
                           preferred_element_type=jnp.float32)


_mid = pl.pallas_call(
    _mid_body,
    grid=(NP // BLK,),
    in_specs=[
        pl.BlockSpec((1, BLK, D), lambda i: (0, i, 0)),
        pl.BlockSpec((1, BLK, D), lambda i: (1, i, 0)),
        pl.BlockSpec((BLK, D), lambda i: (i, 0)),
        pl.BlockSpec((BLK, D), lambda i: (i, 0)),
        pl.BlockSpec((1, D), lambda i: (0, 0)),
        pl.BlockSpec((D, D), lambda i: (0, 0)),
    ],
    out_specs=pl.BlockSpec((BLK, D), lambda i: (i, 0)),
    out_shape=jax.ShapeDtypeStruct((NP, D), jnp.float32),
)


def _last_body(p0_ref, p1_ref, h_ref, db_ref, b_ref, wl_ref, bl_ref, out_ref):
    sv = db_ref[...] * (p0_ref[0] + p1_ref[0] + h_ref[...]) + b_ref[...]
    out_ref[...] = _lrelu(jnp.dot(sv, wl_ref[...],
                                  preferred_element_type=jnp.float32) + bl_ref[...])


_last = pl.pallas_call(
    _last_body,
    grid=(NP // BLK,),
    in_specs=[
        pl.BlockSpec((1, BLK, D), lambda i: (0, i, 0)),
        pl.BlockSpec((1, BLK, D), lambda i: (1, i, 0)),
        pl.BlockSpec((BLK, D), lambda i: (i, 0)),
        pl.BlockSpec((BLK, D), lambda i: (i, 0)),
        pl.BlockSpec((1, D), lambda i: (0, 0)),
        pl.BlockSpec((D, D), lambda i: (0, 0)),
        pl.BlockSpec((1, D), lambda i: (0, 0)),
    ],
    out_specs=pl.BlockSpec((BLK, D), lambda i: (i, 0)),
    out_shape=jax.ShapeDtypeStruct((NP, D), jnp.float32),
)


def _final_body(s0_ref, s1_ref, ct_ref, wf_ref, bf_ref, out_ref):
    ssum = s0_ref[0] + s1_ref[0]
    csum = ct_ref[:, 0:1] + ct_ref[:, 1:2]
    hm = ssum / jnp.maximum(csum, 1.0)
    r = jnp.sum(hm * wf_ref[...], axis=1, keepdims=True) + bf_ref[...]
    out_ref[...] = r[:G, :]


_final = pl.pallas_call(
    _final_body,
    grid=(1,),
    in_specs=[
        pl.BlockSpec((1, 2 * G, D), lambda i: (0, 0, 0)),
        pl.BlockSpec((1, 2 * G, D), lambda i: (1, 0, 0)),
        pl.BlockSpec((2 * G, 2), lambda i: (0, 0)),
        pl.BlockSpec((1, D), lambda i: (0, 0)),
        pl.BlockSpec((1, 1), lambda i: (0, 0)),
    ],
    out_specs=pl.BlockSpec((G, 1), lambda i: (0, 0)),
    out_shape=jax.ShapeDtypeStruct((G, 1), jnp.float32),
)


def kernel(fts, adj, batch, W1, b1, W2, b2, W3, b3, Wl, bl, Wf, bf):
    src, dst = adj[0], adj[1]
    pad_e = EP - E
    srcp = jnp.concatenate(
        [src, jnp.zeros((pad_e,), jnp.int32)]).reshape(ECHUNKS, CH)
    dstp = jnp.concatenate(
        [dst, jnp.full((pad_e,), N, jnp.int32)]).reshape(ECHUNKS, CH)
    xp = jnp.pad(fts, ((0, NP - N), (0, 0)))
    batchp = jnp.concatenate(
        [batch, jnp.full((NP - N,), G, jnp.int32)]).reshape(NPCH, CH)

    degp = _hist(dstp).reshape(NSC, NP)
    db, h = _prep(degp.T, xp, W1)

    p = _edge(h, srcp, dstp)
    h = _mid(p, p, h, db, b1.reshape(1, D), W2)
    p = _edge(h, srcp, dstp)
    h = _mid(p, p, h, db, b2.reshape(1, D), W3)
    p = _edge(h, srcp, dstp)
    xh = _last(p, p, h, db, b3.reshape(1, D), Wl, bl.reshape(1, D))

    ps, pc = _pool(xh, batchp)
    return _final(ps, ps, pc.reshape(NSC, 2 * G).T,
                  Wf.reshape(1, D), bf.reshape(1, 1))
